# baseline (device time: 79266 ns/iter reference)
import jax
import jax.numpy as jnp
from jax import lax
from jax.experimental import pallas as pl
from jax.experimental.pallas import tpu as pltpu

N_CHUNK = 4


def kernel(partial, gamma):
    _, m_total, d = partial.shape
    m_half = m_total // 2
    m_chunk = m_half // N_CHUNK

    def body(
        p_ref,
        g_ref,
        out_ref,
        local_ref,
        stage_ref,
        res_ref,
        send_q_ref,
        recv_q_ref,
        send_s_ref,
        recv_s_ref,
        local_sems,
        stage_sems,
        out_sems,
        send_q_sems,
        recv_q_sems,
        send_s_sems,
        recv_s_sems,
    ):
        my_x = lax.axis_index("x")
        my_y = lax.axis_index("y")
        my_z = lax.axis_index("z")
        peer = (1 - my_x, my_y, my_z)

        my_row0 = my_x * m_half
        peer_row0 = (1 - my_x) * m_half

        def stage_copy(k):
            return pltpu.make_async_copy(
                p_ref.at[0, pl.ds(peer_row0 + k * m_chunk, m_chunk), :],
                stage_ref.at[k % 2],
                stage_sems.at[k % 2],
            )

        def local_copy(k):
            return pltpu.make_async_copy(
                p_ref.at[0, pl.ds(my_row0 + k * m_chunk, m_chunk), :],
                local_ref.at[pl.ds(k * m_chunk, m_chunk)],
                local_sems.at[k],
            )

        stage_copy(0).start()
        local_copy(0).start()

        barrier = pltpu.get_barrier_semaphore()
        pl.semaphore_signal(
            barrier, inc=1, device_id=peer, device_id_type=pl.DeviceIdType.MESH
        )
        pl.semaphore_wait(barrier, 1)

        q_rdmas = []
        s_rdmas = []
        out_dmas = []

        def quant_send(k):
            if k + 1 < N_CHUNK:
                stage_copy(k + 1).start()
                local_copy(k + 1).start()
            stage_copy(k).wait()
            chunk = stage_ref[k % 2, :, :]
            scale = jnp.max(jnp.abs(chunk), axis=-1, keepdims=True) * (1.0 / 127.0)
            scale = jnp.maximum(scale, 1e-30)
            send_s_ref[k, :, :] = scale
            send_q_ref[k, :, :] = jnp.round(chunk * (1.0 / scale)).astype(jnp.int8)
            q_rdma = pltpu.make_async_remote_copy(
                src_ref=send_q_ref.at[k],
                dst_ref=recv_q_ref.at[k],
                send_sem=send_q_sems.at[k],
                recv_sem=recv_q_sems.at[k],
                device_id=peer,
                device_id_type=pl.DeviceIdType.MESH,
            )
            q_rdma.start()
            s_rdma = pltpu.make_async_remote_copy(
                src_ref=send_s_ref.at[k],
                dst_ref=recv_s_ref.at[k],
                send_sem=send_s_sems.at[k],
                recv_sem=recv_s_sems.at[k],
                device_id=peer,
                device_id_type=pl.DeviceIdType.MESH,
            )
            s_rdma.start()
            q_rdmas.append(q_rdma)
            s_rdmas.append(s_rdma)

        def consume(k):
            q_rdmas[k].wait_recv()
            s_rdmas[k].wait_recv()
            local_copy(k).wait()
            rows = pl.ds(k * m_chunk, m_chunk)
            b = recv_q_ref[k, :, :].astype(jnp.float32) * recv_s_ref[k, :, :]
            y = local_ref[rows, :] + b
            ms = jnp.mean(y * y, axis=-1, keepdims=True)
            if k >= 2:
                out_dmas[k - 2].wait()
            res_ref[k % 2, :, :] = y * lax.rsqrt(ms + 1e-6) * g_ref[:, :]
            out_dma = pltpu.make_async_copy(
                res_ref.at[k % 2],
                out_ref.at[pl.ds(k * m_chunk, m_chunk)],
                out_sems.at[k],
            )
            out_dma.start()
            out_dmas.append(out_dma)

        quant_send(0)
        for k in range(1, N_CHUNK):
            quant_send(k)
            consume(k - 1)
        consume(N_CHUNK - 1)

        for k in range(N_CHUNK):
            q_rdmas[k].wait_send()
            s_rdmas[k].wait_send()
        out_dmas[N_CHUNK - 2].wait()
        out_dmas[N_CHUNK - 1].wait()

    return pl.pallas_call(
        body,
        out_shape=jax.ShapeDtypeStruct((m_half, d), jnp.float32),
        in_specs=[
            pl.BlockSpec(memory_space=pl.ANY),
            pl.BlockSpec(memory_space=pltpu.VMEM),
        ],
        out_specs=pl.BlockSpec(memory_space=pl.ANY),
        scratch_shapes=[
            pltpu.VMEM((m_half, d), jnp.float32),
            pltpu.VMEM((2, m_chunk, d), jnp.float32),
            pltpu.VMEM((2, m_chunk, d), jnp.float32),
            pltpu.VMEM((N_CHUNK, m_chunk, d), jnp.int8),
            pltpu.VMEM((N_CHUNK, m_chunk, d), jnp.int8),
            pltpu.VMEM((N_CHUNK, m_chunk, 1), jnp.float32),
            pltpu.VMEM((N_CHUNK, m_chunk, 1), jnp.float32),
            pltpu.SemaphoreType.DMA((N_CHUNK,)),
            pltpu.SemaphoreType.DMA((2,)),
            pltpu.SemaphoreType.DMA((N_CHUNK,)),
            pltpu.SemaphoreType.DMA((N_CHUNK,)),
            pltpu.SemaphoreType.DMA((N_CHUNK,)),
            pltpu.SemaphoreType.DMA((N_CHUNK,)),
            pltpu.SemaphoreType.DMA((N_CHUNK,)),
        ],
        compiler_params=pltpu.CompilerParams(
            collective_id=0, vmem_limit_bytes=60 * 1024 * 1024
        ),
    )(partial, gamma.reshape(1, -1))
